# baseline (speedup 1.0000x reference)
import jax
import jax.numpy as jnp
from jax.experimental import pallas as pl
from jax.experimental.pallas import tpu as pltpu


def _make_body(batch, bs):
    def _body(w_ref, o_ref, sem):
        s = pl.program_id(0)
        copies = [
            pltpu.make_async_copy(w_ref, o_ref.at[b, pl.ds(s * bs, bs)], sem)
            for b in range(batch)
        ]
        for c in copies:
            c.start()
        for c in copies:
            c.wait()
    return _body


def kernel(tokens, W_pos):
    batch, seq_len = tokens.shape
    d_model = W_pos.shape[1]
    bs = 512
    grid = (seq_len // bs,)
    return pl.pallas_call(
        _make_body(batch, bs),
        grid=grid,
        in_specs=[pl.BlockSpec((bs, d_model), lambda s: (s, 0))],
        out_specs=pl.BlockSpec(memory_space=pl.ANY),
        out_shape=jax.ShapeDtypeStruct((batch, seq_len, d_model), W_pos.dtype),
        scratch_shapes=[pltpu.SemaphoreType.DMA],
    )(W_pos)
